# baseline (speedup 1.0000x reference)
import functools

import jax
import jax.numpy as jnp
from jax import lax
from jax.experimental import pallas as pl
from jax.experimental.pallas import tpu as pltpu
from jax.experimental.pallas import tpu_sc as plsc

NBUF = 8
INFLIGHT = 6


@functools.lru_cache(maxsize=None)
def _build(num_rows, row_len, dim):
    mesh = plsc.VectorSubcoreMesh(core_axis_name="c", subcore_axis_name="s")
    nc, ns = mesh.num_cores, mesh.num_subcores
    nw = nc * ns
    assert num_rows % nw == 0
    rows_per_w = num_rows // nw
    assert rows_per_w % NBUF == 0 and rows_per_w >= NBUF

    @functools.partial(
        pl.kernel,
        out_type=jax.ShapeDtypeStruct((num_rows, row_len, dim), jnp.float32),
        mesh=mesh,
        scratch_types=[
            pltpu.VMEM((rows_per_w, row_len), jnp.int32),
            pltpu.VMEM((NBUF, row_len, dim), jnp.float32),
        ]
        + [pltpu.SemaphoreType.DMA] * (2 * NBUF),
        compiler_params=pltpu.CompilerParams(use_tc_tiling_on_sc=False),
    )
    def emb(idx_hbm, table_hbm, out_hbm, idx_v, rows_v, *sems):
        gsems, ssems = sems[:NBUF], sems[NBUF:]
        wid = lax.axis_index("s") * nc + lax.axis_index("c")
        base = wid * rows_per_w
        pltpu.sync_copy(idx_hbm.at[pl.ds(base, rows_per_w)], idx_v)
        for b in range(INFLIGHT):
            pltpu.async_copy(table_hbm.at[idx_v.at[b]], rows_v.at[b], gsems[b])

        @pl.loop(0, rows_per_w, step=NBUF)
        def _(g):
            for b in range(NBUF):
                j = g + b
                pltpu.make_async_copy(
                    table_hbm.at[idx_v.at[b]], rows_v.at[b], gsems[b]
                ).wait()
                pltpu.async_copy(rows_v.at[b], out_hbm.at[base + j], ssems[b])
                nj = j + INFLIGHT
                sb = (b + INFLIGHT) % NBUF

                @pl.when(nj < rows_per_w)
                def _():
                    @pl.when(nj >= NBUF)
                    def _():
                        pltpu.make_async_copy(
                            rows_v.at[sb], out_hbm.at[base], ssems[sb]
                        ).wait()

                    pltpu.async_copy(
                        table_hbm.at[idx_v.at[nj]], rows_v.at[sb], gsems[sb]
                    )

        for b in range(NBUF):
            pltpu.make_async_copy(
                rows_v.at[b], out_hbm.at[base], ssems[b]
            ).wait()

    return emb


def kernel(token_ids, weights):
    emb = _build(token_ids.shape[0], token_ids.shape[1], weights.shape[1])
    return emb(token_ids, weights)
